# baseline (speedup 1.0000x reference)
import functools

import jax
import jax.numpy as jnp
from jax import lax
from jax.experimental import pallas as pl
from jax.experimental.pallas import tpu as pltpu
from jax.experimental.pallas import tpu_sc as plsc

_HIDDEN = 128
_EPS = 1e-12
_WINDOW = 128


def _sc_gather3(wl_table, pos_table, hop_table, wl_i, pos_i, hop_i):
    n = wl_i.shape[0]
    d = wl_table.shape[1]
    mesh = plsc.VectorSubcoreMesh(core_axis_name="c", subcore_axis_name="s")
    n_workers = mesh.num_cores * mesh.num_subcores
    rows_per_w = n // n_workers
    n_win = rows_per_w // _WINDOW
    assert rows_per_w % _WINDOW == 0

    out_sds = jax.ShapeDtypeStruct((n, d), jnp.float32)

    @functools.partial(
        pl.kernel,
        out_type=[out_sds, out_sds, out_sds],
        mesh=mesh,
        scratch_types=[
            pltpu.VMEM((_WINDOW,), jnp.int32),
            pltpu.VMEM((_WINDOW,), jnp.int32),
            pltpu.VMEM((_WINDOW,), jnp.int32),
            pltpu.VMEM((_WINDOW, d), jnp.float32),
            pltpu.VMEM((_WINDOW, d), jnp.float32),
            pltpu.VMEM((_WINDOW, d), jnp.float32),
            pltpu.SemaphoreType.DMA,
        ],
    )
    def sck(wl_t, pos_t, hop_t, wl_idx, pos_idx, hop_idx, o1, o2, o3,
            i1_v, i2_v, i3_v, r1_v, r2_v, r3_v, sem):
        wid = lax.axis_index("s") * mesh.num_cores + lax.axis_index("c")
        base = wid * rows_per_w

        @pl.loop(0, n_win)
        def _(t):
            off = base + t * _WINDOW
            pltpu.sync_copy(wl_idx.at[pl.ds(off, _WINDOW)], i1_v)
            pltpu.sync_copy(pos_idx.at[pl.ds(off, _WINDOW)], i2_v)
            pltpu.sync_copy(hop_idx.at[pl.ds(off, _WINDOW)], i3_v)
            pltpu.async_copy(wl_t.at[i1_v], r1_v, sem).wait()
            pltpu.async_copy(pos_t.at[i2_v], r2_v, sem).wait()
            pltpu.async_copy(hop_t.at[i3_v], r3_v, sem).wait()
            pltpu.sync_copy(r1_v, o1.at[pl.ds(off, _WINDOW)])
            pltpu.sync_copy(r2_v, o2.at[pl.ds(off, _WINDOW)])
            pltpu.sync_copy(r3_v, o3.at[pl.ds(off, _WINDOW)])

    return sck(wl_table, pos_table, hop_table, wl_i, pos_i, hop_i)


def _tc_body(raw_ref, g1_ref, g2_ref, g3_ref, w_ref, b_ref, gamma_ref,
             beta_ref, o_ref):
    bt, s, d = raw_ref.shape
    x2 = raw_ref[...].reshape(bt * s, d)
    x = jnp.dot(x2, w_ref[...], preferred_element_type=jnp.float32)
    e = x + b_ref[...] + g1_ref[...] + g2_ref[...] + g3_ref[...]
    mean = jnp.mean(e, axis=-1, keepdims=True)
    c = e - mean
    var = jnp.mean(c * c, axis=-1, keepdims=True)
    o = c * lax.rsqrt(var + _EPS) * gamma_ref[...] + beta_ref[...]
    o_ref[...] = o.reshape(bt, s, d)


def _tc_fuse(raw, g1, g2, g3, w, b, gamma, beta, tile_b):
    batch, s, d = raw.shape
    grid = (batch // tile_b,)
    raw_spec = pl.BlockSpec((tile_b, s, d), lambda i: (i, 0, 0))
    row_spec = pl.BlockSpec((tile_b * s, d), lambda i: (i, 0))
    full_spec = pl.BlockSpec((d, d), lambda i: (0, 0))
    vec_spec = pl.BlockSpec((1, d), lambda i: (0, 0))
    return pl.pallas_call(
        _tc_body,
        grid=grid,
        in_specs=[raw_spec, row_spec, row_spec, row_spec, full_spec,
                  vec_spec, vec_spec, vec_spec],
        out_specs=raw_spec,
        out_shape=jax.ShapeDtypeStruct((batch, s, d), jnp.float32),
        compiler_params=pltpu.CompilerParams(
            dimension_semantics=("parallel",)),
    )(raw, g1, g2, g3, w, b.reshape(1, d), gamma.reshape(1, d),
      beta.reshape(1, d))


def kernel(raw_features, wl_role_ids, init_pos_ids, hop_dis_ids, W, b,
           wl_table, pos_table, hop_table, gamma, beta):
    batch, seq, x_size = raw_features.shape
    wl_i = wl_role_ids.reshape(-1).astype(jnp.int32)
    pos_i = init_pos_ids.reshape(-1).astype(jnp.int32)
    hop_i = hop_dis_ids.reshape(-1).astype(jnp.int32)

    g1, g2, g3 = _sc_gather3(wl_table, pos_table, hop_table, wl_i, pos_i, hop_i)
    return _tc_fuse(raw_features, g1, g2, g3, W, b, gamma, beta, tile_b=64)


# speedup vs baseline: 1.2495x; 1.2495x over previous
import functools

import jax
import jax.numpy as jnp
from jax import lax
from jax.experimental import pallas as pl
from jax.experimental.pallas import tpu as pltpu
from jax.experimental.pallas import tpu_sc as plsc

_HIDDEN = 128
_EPS = 1e-12
_WINDOW = 128


def _sc_gather3(wl_table, pos_table, hop_table, wl_i, pos_i, hop_i):
    n = wl_i.shape[0]
    d = wl_table.shape[1]
    mesh = plsc.VectorSubcoreMesh(core_axis_name="c", subcore_axis_name="s")
    n_workers = mesh.num_cores * mesh.num_subcores
    rows_per_w = n // n_workers
    n_win = rows_per_w // _WINDOW
    assert rows_per_w % _WINDOW == 0

    out_sds = jax.ShapeDtypeStruct((n, d), jnp.float32)

    vmem_idx = pltpu.VMEM((_WINDOW,), jnp.int32)
    vmem_rows = pltpu.VMEM((_WINDOW, d), jnp.float32)

    @functools.partial(
        pl.kernel,
        out_type=[out_sds, out_sds, out_sds],
        mesh=mesh,
        scratch_types=[
            vmem_idx, vmem_idx, vmem_idx, vmem_idx, vmem_idx, vmem_idx,
            vmem_rows, vmem_rows, vmem_rows, vmem_rows, vmem_rows, vmem_rows,
            pltpu.SemaphoreType.DMA, pltpu.SemaphoreType.DMA,
            pltpu.SemaphoreType.DMA, pltpu.SemaphoreType.DMA,
            pltpu.SemaphoreType.DMA, pltpu.SemaphoreType.DMA,
        ],
    )
    def sck(wl_t, pos_t, hop_t, wl_idx, pos_idx, hop_idx, o1, o2, o3,
            i00, i01, i02, i10, i11, i12,
            g00, g01, g02, g10, g11, g12,
            semi0, semi1, semg0, semg1, semo0, semo1):
        tbl = (wl_t, pos_t, hop_t)
        idx = (wl_idx, pos_idx, hop_idx)
        out = (o1, o2, o3)
        ic = ((i00, i01, i02), (i10, i11, i12))
        gb = ((g00, g01, g02), (g10, g11, g12))
        semi = (semi0, semi1)
        semg = (semg0, semg1)
        semo = (semo0, semo1)

        wid = lax.axis_index("s") * mesh.num_cores + lax.axis_index("c")
        base = wid * rows_per_w

        def fire_idx(w, p):
            off = base + w * _WINDOW
            for k in range(3):
                pltpu.async_copy(idx[k].at[pl.ds(off, _WINDOW)], ic[p][k],
                                 semi[p])

        def drain_idx(p):
            for k in range(3):
                pltpu.make_async_copy(idx[k].at[pl.ds(base, _WINDOW)],
                                      ic[p][k], semi[p]).wait()

        def fire_gathers(p):
            for k in range(3):
                pltpu.async_copy(tbl[k].at[ic[p][k]], gb[p][k], semg[p])

        def drain_gathers(p):
            for k in range(3):
                pltpu.make_async_copy(tbl[k].at[pl.ds(0, _WINDOW)],
                                      gb[p][k], semg[p]).wait()

        def fire_outputs(w, p):
            off = base + w * _WINDOW
            for k in range(3):
                pltpu.async_copy(gb[p][k], out[k].at[pl.ds(off, _WINDOW)],
                                 semo[p])

        def drain_outputs(p):
            for k in range(3):
                pltpu.make_async_copy(tbl[k].at[pl.ds(0, _WINDOW)],
                                      gb[p][k], semo[p]).wait()

        fire_idx(0, 0)
        drain_idx(0)
        fire_gathers(0)
        fire_idx(1, 1)

        @pl.loop(0, n_win // 2)
        def _(j):
            for b in (0, 1):
                w = 2 * j + b
                nb = 1 - b

                @pl.when(w >= 1)
                def _():
                    drain_outputs(nb)

                @pl.when(w + 1 < n_win)
                def _():
                    drain_idx(nb)
                    fire_gathers(nb)

                drain_gathers(b)

                @pl.when(w + 2 < n_win)
                def _():
                    fire_idx(w + 2, b)

                fire_outputs(w, b)

        drain_outputs((n_win - 1) % 2)

    return sck(wl_table, pos_table, hop_table, wl_i, pos_i, hop_i)


def _tc_body(raw_ref, g1_ref, g2_ref, g3_ref, w_ref, b_ref, gamma_ref,
             beta_ref, o_ref):
    bt, s, d = raw_ref.shape
    x2 = raw_ref[...].reshape(bt * s, d)
    x = jnp.dot(x2, w_ref[...], preferred_element_type=jnp.float32)
    e = x + b_ref[...] + g1_ref[...] + g2_ref[...] + g3_ref[...]
    mean = jnp.mean(e, axis=-1, keepdims=True)
    c = e - mean
    var = jnp.mean(c * c, axis=-1, keepdims=True)
    o = c * lax.rsqrt(var + _EPS) * gamma_ref[...] + beta_ref[...]
    o_ref[...] = o.reshape(bt, s, d)


def _tc_fuse(raw, g1, g2, g3, w, b, gamma, beta, tile_b):
    batch, s, d = raw.shape
    grid = (batch // tile_b,)
    raw_spec = pl.BlockSpec((tile_b, s, d), lambda i: (i, 0, 0))
    row_spec = pl.BlockSpec((tile_b * s, d), lambda i: (i, 0))
    full_spec = pl.BlockSpec((d, d), lambda i: (0, 0))
    vec_spec = pl.BlockSpec((1, d), lambda i: (0, 0))
    return pl.pallas_call(
        _tc_body,
        grid=grid,
        in_specs=[raw_spec, row_spec, row_spec, row_spec, full_spec,
                  vec_spec, vec_spec, vec_spec],
        out_specs=raw_spec,
        out_shape=jax.ShapeDtypeStruct((batch, s, d), jnp.float32),
        compiler_params=pltpu.CompilerParams(
            dimension_semantics=("parallel",)),
    )(raw, g1, g2, g3, w, b.reshape(1, d), gamma.reshape(1, d),
      beta.reshape(1, d))


def kernel(raw_features, wl_role_ids, init_pos_ids, hop_dis_ids, W, b,
           wl_table, pos_table, hop_table, gamma, beta):
    batch, seq, x_size = raw_features.shape
    wl_i = wl_role_ids.reshape(-1).astype(jnp.int32)
    pos_i = init_pos_ids.reshape(-1).astype(jnp.int32)
    hop_i = hop_dis_ids.reshape(-1).astype(jnp.int32)

    g1, g2, g3 = _sc_gather3(wl_table, pos_table, hop_table, wl_i, pos_i, hop_i)
    return _tc_fuse(raw_features, g1, g2, g3, W, b, gamma, beta, tile_b=64)


# speedup vs baseline: 1.3811x; 1.1053x over previous
import functools

import jax
import jax.numpy as jnp
from jax import lax
from jax.experimental import pallas as pl
from jax.experimental.pallas import tpu as pltpu
from jax.experimental.pallas import tpu_sc as plsc

_HIDDEN = 128
_EPS = 1e-12
_WINDOW = 128


def _sc_gather3(wl_table, pos_table, hop_table, wl_i, pos_i, hop_i):
    n = wl_i.shape[0]
    d = wl_table.shape[1]
    mesh = plsc.VectorSubcoreMesh(core_axis_name="c", subcore_axis_name="s")
    n_workers = mesh.num_cores * mesh.num_subcores
    rows_per_w = n // n_workers
    n_win = rows_per_w // _WINDOW
    assert rows_per_w % _WINDOW == 0

    out_sds = jax.ShapeDtypeStruct((n, d), jnp.float32)

    vmem_idx = pltpu.VMEM((_WINDOW,), jnp.int32)
    vmem_rows = pltpu.VMEM((_WINDOW, d), jnp.float32)

    @functools.partial(
        pl.kernel,
        out_type=[out_sds, out_sds, out_sds],
        mesh=mesh,
        scratch_types=[
            vmem_idx, vmem_idx, vmem_idx, vmem_idx, vmem_idx, vmem_idx,
            vmem_rows, vmem_rows, vmem_rows, vmem_rows, vmem_rows, vmem_rows,
            pltpu.SemaphoreType.DMA, pltpu.SemaphoreType.DMA,
            pltpu.SemaphoreType.DMA, pltpu.SemaphoreType.DMA,
            pltpu.SemaphoreType.DMA, pltpu.SemaphoreType.DMA,
        ],
    )
    def sck(wl_t, pos_t, hop_t, wl_idx, pos_idx, hop_idx, o1, o2, o3,
            i00, i01, i02, i10, i11, i12,
            g00, g01, g02, g10, g11, g12,
            semi0, semi1, semg0, semg1, semo0, semo1):
        tbl = (wl_t, pos_t, hop_t)
        idx = (wl_idx, pos_idx, hop_idx)
        out = (o1, o2, o3)
        ic = ((i00, i01, i02), (i10, i11, i12))
        gb = ((g00, g01, g02), (g10, g11, g12))
        semi = (semi0, semi1)
        semg = (semg0, semg1)
        semo = (semo0, semo1)

        wid = lax.axis_index("s") * mesh.num_cores + lax.axis_index("c")
        base = wid * rows_per_w

        def fire_idx(w, p):
            off = base + w * _WINDOW
            for k in range(3):
                pltpu.async_copy(idx[k].at[pl.ds(off, _WINDOW)], ic[p][k],
                                 semi[p])

        def drain_idx(p):
            for k in range(3):
                pltpu.make_async_copy(idx[k].at[pl.ds(base, _WINDOW)],
                                      ic[p][k], semi[p]).wait()

        def fire_gathers(p):
            for k in range(3):
                pltpu.async_copy(tbl[k].at[ic[p][k]], gb[p][k], semg[p])

        def drain_gathers(p):
            for k in range(3):
                pltpu.make_async_copy(tbl[k].at[pl.ds(0, _WINDOW)],
                                      gb[p][k], semg[p]).wait()

        def fire_outputs(w, p):
            off = base + w * _WINDOW
            for k in range(3):
                pltpu.async_copy(gb[p][k], out[k].at[pl.ds(off, _WINDOW)],
                                 semo[p])

        def drain_outputs(p):
            for k in range(3):
                pltpu.make_async_copy(tbl[k].at[pl.ds(0, _WINDOW)],
                                      gb[p][k], semo[p]).wait()

        fire_idx(0, 0)
        drain_idx(0)
        fire_gathers(0)
        fire_idx(1, 1)

        @pl.loop(0, n_win // 2)
        def _(j):
            for b in (0, 1):
                w = 2 * j + b
                nb = 1 - b

                @pl.when(w >= 1)
                def _():
                    drain_outputs(nb)

                @pl.when(w + 1 < n_win)
                def _():
                    drain_idx(nb)
                    fire_gathers(nb)

                drain_gathers(b)

                @pl.when(w + 2 < n_win)
                def _():
                    fire_idx(w + 2, b)

                fire_outputs(w, b)

        drain_outputs((n_win - 1) % 2)

    return sck(wl_table, pos_table, hop_table, wl_i, pos_i, hop_i)


def _tc_body(raw_ref, g1_ref, g2_ref, g3_ref, w_ref, b_ref, gamma_ref,
             beta_ref, o_ref):
    bt, s, d = raw_ref.shape
    x2 = raw_ref[...].reshape(bt * s, d)
    x = jnp.dot(x2, w_ref[...], preferred_element_type=jnp.float32)
    e = x + b_ref[...] + g1_ref[...] + g2_ref[...] + g3_ref[...]
    mean = jnp.mean(e, axis=-1, keepdims=True)
    c = e - mean
    var = jnp.mean(c * c, axis=-1, keepdims=True)
    o = c * lax.rsqrt(var + _EPS) * gamma_ref[...] + beta_ref[...]
    o_ref[...] = o.reshape(bt, s, d)


def _tc_fuse(raw, g1, g2, g3, w, b, gamma, beta, tile_b):
    batch, s, d = raw.shape
    grid = (batch // tile_b,)
    raw_spec = pl.BlockSpec((tile_b, s, d), lambda i: (i, 0, 0))
    row_spec = pl.BlockSpec((tile_b * s, d), lambda i: (i, 0))
    full_spec = pl.BlockSpec((d, d), lambda i: (0, 0))
    vec_spec = pl.BlockSpec((1, d), lambda i: (0, 0))
    return pl.pallas_call(
        _tc_body,
        grid=grid,
        in_specs=[raw_spec, row_spec, row_spec, row_spec, full_spec,
                  vec_spec, vec_spec, vec_spec],
        out_specs=raw_spec,
        out_shape=jax.ShapeDtypeStruct((batch, s, d), jnp.float32),
        compiler_params=pltpu.CompilerParams(
            dimension_semantics=("parallel",)),
    )(raw, g1, g2, g3, w, b.reshape(1, d), gamma.reshape(1, d),
      beta.reshape(1, d))


def kernel(raw_features, wl_role_ids, init_pos_ids, hop_dis_ids, W, b,
           wl_table, pos_table, hop_table, gamma, beta):
    batch, seq, x_size = raw_features.shape
    wl_i = wl_role_ids.reshape(-1).astype(jnp.int32)
    pos_i = init_pos_ids.reshape(-1).astype(jnp.int32)
    hop_i = hop_dis_ids.reshape(-1).astype(jnp.int32)

    g1, g2, g3 = _sc_gather3(wl_table, pos_table, hop_table, wl_i, pos_i, hop_i)
    return _tc_fuse(raw_features, g1, g2, g3, W, b, gamma, beta, tile_b=256)


# speedup vs baseline: 1.7571x; 1.2722x over previous
import functools

import jax
import jax.numpy as jnp
from jax import lax
from jax.experimental import pallas as pl
from jax.experimental.pallas import tpu as pltpu
from jax.experimental.pallas import tpu_sc as plsc

_HIDDEN = 128
_EPS = 1e-12
_WINDOW = 128


def _sc_gather3(wl_table, pos_table, hop_table, wl_i, pos_i, hop_i):
    n = wl_i.shape[0]
    d = wl_table.shape[1]
    mesh = plsc.VectorSubcoreMesh(core_axis_name="c", subcore_axis_name="s")
    n_workers = mesh.num_cores * mesh.num_subcores
    rows_per_w = n // n_workers
    n_win = rows_per_w // _WINDOW
    assert rows_per_w % _WINDOW == 0

    out_sds = jax.ShapeDtypeStruct((n, d), jnp.float32)

    vmem_idx = pltpu.VMEM((_WINDOW,), jnp.int32)
    vmem_rows = pltpu.VMEM((_WINDOW, d), jnp.float32)

    @functools.partial(
        pl.kernel,
        out_type=out_sds,
        mesh=mesh,
        scratch_types=[
            vmem_idx, vmem_idx, vmem_idx, vmem_idx, vmem_idx, vmem_idx,
            vmem_rows, vmem_rows, vmem_rows, vmem_rows, vmem_rows, vmem_rows,
            pltpu.SemaphoreType.DMA, pltpu.SemaphoreType.DMA,
            pltpu.SemaphoreType.DMA, pltpu.SemaphoreType.DMA,
            pltpu.SemaphoreType.DMA, pltpu.SemaphoreType.DMA,
        ],
    )
    def sck(wl_t, pos_t, hop_t, wl_idx, pos_idx, hop_idx, osum,
            i00, i01, i02, i10, i11, i12,
            g00, g01, g02, g10, g11, g12,
            semi0, semi1, semg0, semg1, semo0, semo1):
        tbl = (wl_t, pos_t, hop_t)
        idx = (wl_idx, pos_idx, hop_idx)
        ic = ((i00, i01, i02), (i10, i11, i12))
        gb = ((g00, g01, g02), (g10, g11, g12))
        semi = (semi0, semi1)
        semg = (semg0, semg1)
        semo = (semo0, semo1)

        wid = lax.axis_index("s") * mesh.num_cores + lax.axis_index("c")
        base = wid * rows_per_w

        def fire_idx(w, p):
            off = base + w * _WINDOW
            for k in range(3):
                pltpu.async_copy(idx[k].at[pl.ds(off, _WINDOW)], ic[p][k],
                                 semi[p])

        def drain_idx(p):
            for k in range(3):
                pltpu.make_async_copy(idx[k].at[pl.ds(base, _WINDOW)],
                                      ic[p][k], semi[p]).wait()

        def fire_gathers(p):
            for k in range(3):
                pltpu.async_copy(tbl[k].at[ic[p][k]], gb[p][k], semg[p])

        def drain_gathers(p):
            for k in range(3):
                pltpu.make_async_copy(tbl[k].at[pl.ds(0, _WINDOW)],
                                      gb[p][k], semg[p]).wait()

        def sum_bufs(p):
            a0, a1, a2 = gb[p]

            @pl.loop(0, _WINDOW)
            def _(r):
                for c in range(0, d, 16):
                    slc = (pl.ds(r, 1), pl.ds(c, 16))
                    a0.at[slc][...] = (
                        a0.at[slc][...] + a1.at[slc][...] + a2.at[slc][...])

        def fire_outputs(w, p):
            off = base + w * _WINDOW
            pltpu.async_copy(gb[p][0], osum.at[pl.ds(off, _WINDOW)], semo[p])

        def drain_outputs(p):
            pltpu.make_async_copy(tbl[0].at[pl.ds(0, _WINDOW)],
                                  gb[p][0], semo[p]).wait()

        fire_idx(0, 0)
        drain_idx(0)
        fire_gathers(0)
        fire_idx(1, 1)

        @pl.loop(0, n_win // 2)
        def _(j):
            for b in (0, 1):
                w = 2 * j + b
                nb = 1 - b

                @pl.when(w >= 1)
                def _():
                    drain_outputs(nb)

                @pl.when(w + 1 < n_win)
                def _():
                    drain_idx(nb)
                    fire_gathers(nb)

                drain_gathers(b)

                @pl.when(w + 2 < n_win)
                def _():
                    fire_idx(w + 2, b)

                sum_bufs(b)
                fire_outputs(w, b)

        drain_outputs((n_win - 1) % 2)

    return sck(wl_table, pos_table, hop_table, wl_i, pos_i, hop_i)


def _tc_body(raw_ref, g_ref, w_ref, b_ref, gamma_ref, beta_ref, o_ref):
    bt, s, d = raw_ref.shape
    x2 = raw_ref[...].reshape(bt * s, d)
    x = jnp.dot(x2, w_ref[...], preferred_element_type=jnp.float32)
    e = x + b_ref[...] + g_ref[...]
    mean = jnp.mean(e, axis=-1, keepdims=True)
    c = e - mean
    var = jnp.mean(c * c, axis=-1, keepdims=True)
    o = c * lax.rsqrt(var + _EPS) * gamma_ref[...] + beta_ref[...]
    o_ref[...] = o.reshape(bt, s, d)


def _tc_fuse(raw, g, w, b, gamma, beta, tile_b):
    batch, s, d = raw.shape
    grid = (batch // tile_b,)
    raw_spec = pl.BlockSpec((tile_b, s, d), lambda i: (i, 0, 0))
    row_spec = pl.BlockSpec((tile_b * s, d), lambda i: (i, 0))
    full_spec = pl.BlockSpec((d, d), lambda i: (0, 0))
    vec_spec = pl.BlockSpec((1, d), lambda i: (0, 0))
    return pl.pallas_call(
        _tc_body,
        grid=grid,
        in_specs=[raw_spec, row_spec, full_spec,
                  vec_spec, vec_spec, vec_spec],
        out_specs=raw_spec,
        out_shape=jax.ShapeDtypeStruct((batch, s, d), jnp.float32),
        compiler_params=pltpu.CompilerParams(
            dimension_semantics=("parallel",)),
    )(raw, g, w, b.reshape(1, d), gamma.reshape(1, d), beta.reshape(1, d))


def kernel(raw_features, wl_role_ids, init_pos_ids, hop_dis_ids, W, b,
           wl_table, pos_table, hop_table, gamma, beta):
    batch, seq, x_size = raw_features.shape
    wl_i = wl_role_ids.reshape(-1).astype(jnp.int32)
    pos_i = init_pos_ids.reshape(-1).astype(jnp.int32)
    hop_i = hop_dis_ids.reshape(-1).astype(jnp.int32)

    g = _sc_gather3(wl_table, pos_table, hop_table, wl_i, pos_i, hop_i)
    return _tc_fuse(raw_features, g, W, b, gamma, beta, tile_b=512)
